# baseline (speedup 1.0000x reference)
import functools

import jax
import jax.numpy as jnp
from jax import lax
from jax.experimental import pallas as pl
from jax.experimental.pallas import tpu as pltpu
from jax.experimental.pallas import tpu_sc as plsc

B = 4096
D_IN = 768
D_H = 128
D_Z = 256
K = 8192

B_BLK = 1024
NB = B // B_BLK


def _encode_kernel(x_ref, w1_ref, b1_ref, w2_ref, b2_ref, cb_ref,
                   idx_ref, cbn_ref):
    @pl.when(pl.program_id(0) == 0)
    def _():
        cb = cb_ref[...]
        cbn_ref[...] = jnp.sum(cb * cb, axis=1)[None, :]

    x = x_ref[...]
    h = lax.dot_general(x, w1_ref[...], (((1,), (1,)), ((), ())),
                        preferred_element_type=jnp.float32)
    h = jnp.maximum(h + b1_ref[...], 0.0)
    z = lax.dot_general(h, w2_ref[...], (((1,), (1,)), ((), ())),
                        preferred_element_type=jnp.float32)
    z = z + b2_ref[...]
    a2 = jnp.sum(z * z, axis=1, keepdims=True)
    zm2 = z * -2.0
    ab = lax.dot_general(zm2, cb_ref[...], (((1,), (1,)), ((), ())),
                         preferred_element_type=jnp.float32)
    d2 = (a2 + cbn_ref[...]) + ab
    idx_ref[...] = jnp.argmin(d2, axis=1).astype(jnp.int32)


_encode = pl.pallas_call(
    _encode_kernel,
    grid=(NB,),
    in_specs=[
        pl.BlockSpec((B_BLK, D_IN), lambda i: (i, 0)),
        pl.BlockSpec((D_H, D_IN), lambda i: (0, 0)),
        pl.BlockSpec((1, D_H), lambda i: (0, 0)),
        pl.BlockSpec((D_Z, D_H), lambda i: (0, 0)),
        pl.BlockSpec((1, D_Z), lambda i: (0, 0)),
        pl.BlockSpec((K, D_Z), lambda i: (0, 0)),
    ],
    out_specs=pl.BlockSpec((B_BLK,), lambda i: (i,)),
    out_shape=jax.ShapeDtypeStruct((B,), jnp.int32),
    scratch_shapes=[pltpu.VMEM((1, K), jnp.float32)],
)

_NC = 2
_NS = 16
_NW = _NC * _NS
B_PER_W = B // _NW

D_HALF = D_Z // _NC
B_PER_S = B // _NS
K_PER_S = K // _NS


@functools.lru_cache(maxsize=1)
def _make_sc_gather():
    mesh = plsc.VectorSubcoreMesh(core_axis_name="c", subcore_axis_name="s",
                                  num_cores=_NC, num_subcores=_NS)

    @functools.partial(
        pl.kernel,
        mesh=mesh,
        out_type=jax.ShapeDtypeStruct((B, D_Z), jnp.float32),
        scratch_types=[
            pltpu.VMEM((B_PER_S,), jnp.int32),
            pltpu.VMEM((B_PER_S, D_HALF), jnp.float32),
            pltpu.VMEM_SHARED((K, D_HALF), jnp.float32),
            pltpu.SemaphoreType.DMA,
        ],
    )
    def _sc_gather(cb_hbm, idx_hbm, out_hbm, idx_v, rows_v, spmem, sem):
        c = lax.axis_index("c")
        s = lax.axis_index("s")
        col = c * D_HALF
        pltpu.sync_copy(cb_hbm.at[pl.ds(s * K_PER_S, K_PER_S),
                                  pl.ds(col, D_HALF)],
                        spmem.at[pl.ds(s * K_PER_S, K_PER_S)])
        plsc.subcore_barrier()
        base = s * B_PER_S
        pltpu.sync_copy(idx_hbm.at[pl.ds(base, B_PER_S)], idx_v)
        pltpu.async_copy(spmem.at[idx_v], rows_v, sem).wait()
        pltpu.sync_copy(rows_v,
                        out_hbm.at[pl.ds(base, B_PER_S), pl.ds(col, D_HALF)])

    return _sc_gather


def kernel(x, W1, b1, W2, b2, codebook):
    indices = _encode(x, W1, b1.reshape(1, D_H), W2, b2.reshape(1, D_Z),
                      codebook)
    z_q = _make_sc_gather()(codebook, indices)
    return (z_q, indices)


# speedup vs baseline: 1.0293x; 1.0293x over previous
import functools

import jax
import jax.numpy as jnp
from jax import lax
from jax.experimental import pallas as pl
from jax.experimental.pallas import tpu as pltpu
from jax.experimental.pallas import tpu_sc as plsc

B = 4096
D_IN = 768
D_H = 128
D_Z = 256
K = 8192

B_BLK = 512
NB = B // B_BLK


def _encode_kernel(x_ref, w1_ref, b1_ref, w2_ref, b2_ref, cb_ref,
                   idx_ref, cbn_ref):
    @pl.when(pl.program_id(0) == 0)
    def _():
        cb = cb_ref[...]
        cbn_ref[...] = jnp.sum(cb * cb, axis=1)[None, :]

    x = x_ref[...]
    h = lax.dot_general(x, w1_ref[...], (((1,), (1,)), ((), ())),
                        preferred_element_type=jnp.float32)
    h = jnp.maximum(h + b1_ref[...], 0.0)
    z = lax.dot_general(h, w2_ref[...], (((1,), (1,)), ((), ())),
                        preferred_element_type=jnp.float32)
    z = z + b2_ref[...]
    a2 = jnp.sum(z * z, axis=1, keepdims=True)
    zm2 = z * -2.0
    ab = lax.dot_general(zm2, cb_ref[...], (((1,), (1,)), ((), ())),
                         preferred_element_type=jnp.float32)
    d2 = (a2 + cbn_ref[...]) + ab
    idx_ref[...] = jnp.argmin(d2, axis=1).astype(jnp.int32)


_encode = pl.pallas_call(
    _encode_kernel,
    grid=(NB,),
    in_specs=[
        pl.BlockSpec((B_BLK, D_IN), lambda i: (i, 0)),
        pl.BlockSpec((D_H, D_IN), lambda i: (0, 0)),
        pl.BlockSpec((1, D_H), lambda i: (0, 0)),
        pl.BlockSpec((D_Z, D_H), lambda i: (0, 0)),
        pl.BlockSpec((1, D_Z), lambda i: (0, 0)),
        pl.BlockSpec((K, D_Z), lambda i: (0, 0)),
    ],
    out_specs=pl.BlockSpec((B_BLK,), lambda i: (i,)),
    out_shape=jax.ShapeDtypeStruct((B,), jnp.int32),
    scratch_shapes=[pltpu.VMEM((1, K), jnp.float32)],
)

_NC = 2
_NS = 16
_NW = _NC * _NS
B_PER_W = B // _NW

D_HALF = D_Z // _NC
B_PER_S = B // _NS
K_PER_S = K // _NS


@functools.lru_cache(maxsize=1)
def _make_sc_gather():
    mesh = plsc.VectorSubcoreMesh(core_axis_name="c", subcore_axis_name="s",
                                  num_cores=_NC, num_subcores=_NS)

    @functools.partial(
        pl.kernel,
        mesh=mesh,
        out_type=jax.ShapeDtypeStruct((B, D_Z), jnp.float32),
        scratch_types=[
            pltpu.VMEM((B_PER_S,), jnp.int32),
            pltpu.VMEM((B_PER_S, D_HALF), jnp.float32),
            pltpu.VMEM_SHARED((K, D_HALF), jnp.float32),
            pltpu.SemaphoreType.DMA,
        ],
    )
    def _sc_gather(cb_hbm, idx_hbm, out_hbm, idx_v, rows_v, spmem, sem):
        c = lax.axis_index("c")
        s = lax.axis_index("s")
        col = c * D_HALF
        base = s * B_PER_S
        idx_cp = pltpu.async_copy(idx_hbm.at[pl.ds(base, B_PER_S)], idx_v,
                                  sem)
        pltpu.sync_copy(cb_hbm.at[pl.ds(s * K_PER_S, K_PER_S),
                                  pl.ds(col, D_HALF)],
                        spmem.at[pl.ds(s * K_PER_S, K_PER_S)])
        idx_cp.wait()
        plsc.subcore_barrier()
        pltpu.async_copy(spmem.at[idx_v], rows_v, sem).wait()
        pltpu.sync_copy(rows_v,
                        out_hbm.at[pl.ds(base, B_PER_S), pl.ds(col, D_HALF)])

    return _sc_gather


def kernel(x, W1, b1, W2, b2, codebook):
    indices = _encode(x, W1, b1.reshape(1, D_H), W2, b2.reshape(1, D_Z),
                      codebook)
    z_q = _make_sc_gather()(codebook, indices)
    return (z_q, indices)


# speedup vs baseline: 1.0319x; 1.0026x over previous
import functools

import jax
import jax.numpy as jnp
from jax import lax
from jax.experimental import pallas as pl
from jax.experimental.pallas import tpu as pltpu
from jax.experimental.pallas import tpu_sc as plsc

B = 4096
D_IN = 768
D_H = 128
D_Z = 256
K = 8192

B_BLK = 512
NB = B // B_BLK


def _encode_kernel(x_ref, w1_ref, b1_ref, w2_ref, b2_ref, cb_ref,
                   idx_ref, cbn_ref):
    @pl.when(pl.program_id(0) == 0)
    def _():
        cb = cb_ref[...]
        cbn_ref[...] = jnp.sum(cb * cb, axis=1)[None, :]

    x = x_ref[...]
    h = lax.dot_general(x, w1_ref[...], (((1,), (1,)), ((), ())),
                        preferred_element_type=jnp.float32)
    h = jnp.maximum(h + b1_ref[...], 0.0)
    z = lax.dot_general(h, w2_ref[...], (((1,), (1,)), ((), ())),
                        preferred_element_type=jnp.float32)
    z = z + b2_ref[...]
    a2 = jnp.sum(z * z, axis=1, keepdims=True)
    zm2 = z * -2.0
    ab = lax.dot_general(zm2, cb_ref[...], (((1,), (1,)), ((), ())),
                         preferred_element_type=jnp.float32)
    d2 = (a2 + cbn_ref[...]) + ab
    idx_ref[...] = jnp.argmin(d2, axis=1).astype(jnp.int32)


_encode = pl.pallas_call(
    _encode_kernel,
    grid=(NB,),
    in_specs=[
        pl.BlockSpec((B_BLK, D_IN), lambda i: (i, 0)),
        pl.BlockSpec((D_H, D_IN), lambda i: (0, 0)),
        pl.BlockSpec((1, D_H), lambda i: (0, 0)),
        pl.BlockSpec((D_Z, D_H), lambda i: (0, 0)),
        pl.BlockSpec((1, D_Z), lambda i: (0, 0)),
        pl.BlockSpec((K, D_Z), lambda i: (0, 0)),
    ],
    out_specs=pl.BlockSpec((B_BLK,), lambda i: (i,)),
    out_shape=jax.ShapeDtypeStruct((B,), jnp.int32),
    scratch_shapes=[pltpu.VMEM((1, K), jnp.float32)],
)

_NC = 2
_NS = 16
_NW = _NC * _NS
B_PER_W = B // _NW

D_HALF = D_Z // _NC
B_PER_S = B // _NS
K_PER_S = K // _NS


@functools.lru_cache(maxsize=1)
def _make_sc_gather():
    mesh = plsc.VectorSubcoreMesh(core_axis_name="c", subcore_axis_name="s",
                                  num_cores=_NC, num_subcores=_NS)

    @functools.partial(
        pl.kernel,
        mesh=mesh,
        out_type=jax.ShapeDtypeStruct((B, D_Z), jnp.float32),
        scratch_types=[
            pltpu.VMEM((B_PER_S,), jnp.int32),
            pltpu.VMEM((B_PER_S, D_HALF), jnp.float32),
            pltpu.VMEM_SHARED((K, D_HALF), jnp.float32),
            pltpu.SemaphoreType.DMA,
            pltpu.SemaphoreType.DMA,
        ],
    )
    def _sc_gather(cb_hbm, idx_hbm, out_hbm, idx_v, rows_v, spmem, sem,
                   sem_wb):
        c = lax.axis_index("c")
        s = lax.axis_index("s")
        col = c * D_HALF
        base = s * B_PER_S
        idx_cp = pltpu.async_copy(idx_hbm.at[pl.ds(base, B_PER_S)], idx_v,
                                  sem)
        pltpu.sync_copy(cb_hbm.at[pl.ds(s * K_PER_S, K_PER_S),
                                  pl.ds(col, D_HALF)],
                        spmem.at[pl.ds(s * K_PER_S, K_PER_S)])
        idx_cp.wait()
        plsc.subcore_barrier()
        half = B_PER_S // 2
        pltpu.async_copy(spmem.at[idx_v.at[pl.ds(0, half)]],
                         rows_v.at[pl.ds(0, half)], sem).wait()
        wb0 = pltpu.async_copy(
            rows_v.at[pl.ds(0, half)],
            out_hbm.at[pl.ds(base, half), pl.ds(col, D_HALF)], sem_wb)
        pltpu.async_copy(spmem.at[idx_v.at[pl.ds(half, half)]],
                         rows_v.at[pl.ds(half, half)], sem).wait()
        wb1 = pltpu.async_copy(
            rows_v.at[pl.ds(half, half)],
            out_hbm.at[pl.ds(base + half, half), pl.ds(col, D_HALF)], sem_wb)
        wb0.wait()
        wb1.wait()

    return _sc_gather


def kernel(x, W1, b1, W2, b2, codebook):
    indices = _encode(x, W1, b1.reshape(1, D_H), W2, b2.reshape(1, D_Z),
                      codebook)
    z_q = _make_sc_gather()(codebook, indices)
    return (z_q, indices)
